# Initial kernel scaffold; baseline (speedup 1.0000x reference)
import functools

import numpy as np
import jax
import jax.numpy as jnp
from jax import lax
from jax.experimental import pallas as pl
from jax.experimental.pallas import tpu as pltpu
from jax.experimental.pallas import tpu_sc as plsc

N = 10000
E = 320000
D = 128
H = 16
LATENT = 7
PARAMS = 35

NC = 2
NS = 16
NW = NC * NS
SB = 80
EW = E // NW
NB = EW // SB
NROWS = E // SB
RPT = N // NS

ROW_BLK = 1000


def _sc_body(hw, srcm, dstm, wm, out, acc, src_v, dst_v, w_v, rows_v, zb_v,
             sem):
    cid = lax.axis_index("c")
    sid = lax.axis_index("s")
    wid = sid * NC + cid

    def _z(i, c):
        zb_v[i, :] = jnp.zeros((H,), jnp.float32)
        return c

    lax.fori_loop(0, RPT, _z, 0)
    pltpu.sync_copy(zb_v, acc.at[pl.ds(sid * RPT, RPT)])
    plsc.subcore_barrier()

    base = wid * NB
    pltpu.sync_copy(srcm.at[pl.ds(base, NB)], src_v)
    pltpu.sync_copy(dstm.at[pl.ds(base, NB)], dst_v)
    pltpu.sync_copy(wm.at[pl.ds(base, NB)], w_v)

    def _batch(j, c):
        pltpu.async_copy(hw.at[src_v.at[j]], rows_v, sem).wait()
        for e in range(SB):
            rows_v[e, :] = rows_v[e, :] * w_v[j, e]
        pltpu.sync_copy(rows_v, acc.at[dst_v.at[j]], add=True)
        return c

    lax.fori_loop(0, NB, _batch, 0)
    plsc.subcore_barrier()

    pltpu.sync_copy(acc.at[pl.ds(sid * RPT, RPT)],
                    out.at[pl.ds(cid * N + sid * RPT, RPT)])


_sc_aggregate = pl.kernel(
    _sc_body,
    out_type=jax.ShapeDtypeStruct((NC * N, H), jnp.float32),
    mesh=plsc.VectorSubcoreMesh(core_axis_name="c", subcore_axis_name="s",
                                num_cores=NC, num_subcores=NS),
    scratch_types=[
        pltpu.VMEM_SHARED((N, H), jnp.float32),
        pltpu.VMEM((NB, SB), jnp.int32),
        pltpu.VMEM((NB, SB), jnp.int32),
        pltpu.VMEM((NB, SB), jnp.float32),
        pltpu.VMEM((SB, H), jnp.float32),
        pltpu.VMEM((RPT, H), jnp.float32),
        pltpu.SemaphoreType.DMA,
    ],
)


def _mm_body(x_ref, w_ref, o_ref):
    o_ref[...] = jnp.dot(x_ref[...], w_ref[...],
                         preferred_element_type=jnp.float32)


_mm_xw1 = pl.pallas_call(
    _mm_body,
    grid=(N // ROW_BLK,),
    in_specs=[pl.BlockSpec((ROW_BLK, D), lambda i: (i, 0)),
              pl.BlockSpec((D, H), lambda i: (0, 0))],
    out_specs=pl.BlockSpec((ROW_BLK, H), lambda i: (i, 0)),
    out_shape=jax.ShapeDtypeStruct((N, H), jnp.float32),
)


def _mid_body(p_ref, b1_ref, w2_ref, o_ref):
    h = p_ref[0] + p_ref[1] + b1_ref[...]
    h = jnp.maximum(h, 0.0)
    o_ref[...] = jnp.dot(h, w2_ref[...], preferred_element_type=jnp.float32)


_mid = pl.pallas_call(
    _mid_body,
    grid=(N // ROW_BLK,),
    in_specs=[pl.BlockSpec((2, ROW_BLK, H), lambda i: (0, i, 0)),
              pl.BlockSpec((1, H), lambda i: (0, 0)),
              pl.BlockSpec((H, H), lambda i: (0, 0))],
    out_specs=pl.BlockSpec((ROW_BLK, H), lambda i: (i, 0)),
    out_shape=jax.ShapeDtypeStruct((N, H), jnp.float32),
)


def _head_body(p_ref, b2_ref, wd_ref, bd_ref, eps_ref, o_ref):
    h2 = p_ref[0] + p_ref[1] + b2_ref[...]
    params = jnp.dot(h2, wd_ref[...],
                     preferred_element_type=jnp.float32) + bd_ref[...]
    eps = eps_ref[...]
    cols = []
    for i in range(LATENT):
        s = params[:, i:i + 1]
        for j in range(i + 1):
            k = i * (i + 1) // 2 + j
            c = params[:, LATENT + k:LATENT + k + 1]
            if i == j:
                c = jax.nn.softplus(c) + 1e-5
            s = s + c * eps[:, j:j + 1]
        cols.append(s)
    o_ref[...] = jnp.concatenate(cols, axis=1)


_head = pl.pallas_call(
    _head_body,
    grid=(N // ROW_BLK,),
    in_specs=[pl.BlockSpec((2, ROW_BLK, H), lambda i: (0, i, 0)),
              pl.BlockSpec((1, H), lambda i: (0, 0)),
              pl.BlockSpec((H, PARAMS), lambda i: (0, 0)),
              pl.BlockSpec((1, PARAMS), lambda i: (0, 0)),
              pl.BlockSpec((ROW_BLK, LATENT), lambda i: (i, 0))],
    out_specs=pl.BlockSpec((ROW_BLK, LATENT), lambda i: (i, 0)),
    out_shape=jax.ShapeDtypeStruct((N, LATENT), jnp.float32),
)


def kernel(x, edge_index, edge_weight, eps, W1, b1, W2, b2, Wd, bd):
    src = edge_index[0].astype(jnp.int32).reshape(NROWS, SB)
    dst = edge_index[1].astype(jnp.int32).reshape(NROWS, SB)
    ew = edge_weight.reshape(NROWS, SB)

    hw1 = _mm_xw1(x, W1)
    p1 = _sc_aggregate(hw1, src, dst, ew).reshape(2, N, H)
    hw2 = _mid(p1, b1.reshape(1, H), W2)
    p2 = _sc_aggregate(hw2, src, dst, ew).reshape(2, N, H)
    return _head(p2, b2.reshape(1, H), Wd, bd.reshape(1, PARAMS), eps)


# speedup vs baseline: 8.3727x; 8.3727x over previous
import functools

import numpy as np
import jax
import jax.numpy as jnp
from jax import lax
from jax.experimental import pallas as pl
from jax.experimental.pallas import tpu as pltpu
from jax.experimental.pallas import tpu_sc as plsc

N = 10000
E = 320000
D = 128
H = 16
LATENT = 7
PARAMS = 35

NC = 2
NS = 16
NW = NC * NS
SB = 80
EW = E // NW
NB = EW // SB
NROWS = E // SB
NPAD = 10240
RPT = NPAD // NS

ROW_BLK = 1000


def _sc_body(hw, srcm, dstm, wm, out, acc, src_v, dst_v, w_v, rows_v, zb_v,
             sem):
    cid = lax.axis_index("c")
    sid = lax.axis_index("s")
    wid = sid * NC + cid

    def _z(i, c):
        zb_v[i, :] = jnp.zeros((H,), jnp.float32)
        return c

    lax.fori_loop(0, RPT, _z, 0)
    pltpu.sync_copy(zb_v, acc.at[pl.ds(sid * RPT, RPT)])
    plsc.subcore_barrier()

    pltpu.sync_copy(srcm.at[wid], src_v)
    pltpu.sync_copy(dstm.at[wid], dst_v)
    pltpu.sync_copy(wm.at[wid], w_v)

    def _batch(j, c):
        pltpu.async_copy(hw.at[src_v.at[j]], rows_v, sem).wait()
        for g in range(SB // H):
            w16 = w_v[j, pl.ds(g * H, H)]
            for e in range(H):
                rows_v[g * H + e, :] = rows_v[g * H + e, :] * w16[e]
        pltpu.sync_copy(rows_v, acc.at[dst_v.at[j]], add=True)
        return c

    lax.fori_loop(0, NB, _batch, 0)
    plsc.subcore_barrier()

    pltpu.sync_copy(acc.at[pl.ds(sid * RPT, RPT)],
                    out.at[cid, pl.ds(sid * RPT, RPT)])


@functools.lru_cache(maxsize=None)
def _get_sc_aggregate():
  return pl.kernel(
    _sc_body,
    out_type=jax.ShapeDtypeStruct((NC, NPAD, H), jnp.float32),
    mesh=plsc.VectorSubcoreMesh(core_axis_name="c", subcore_axis_name="s",
                                num_cores=NC, num_subcores=NS),
    scratch_types=[
        pltpu.VMEM_SHARED((NPAD, H), jnp.float32),
        pltpu.VMEM((NB, SB), jnp.int32),
        pltpu.VMEM((NB, SB), jnp.int32),
        pltpu.VMEM((NB, SB), jnp.float32),
        pltpu.VMEM((SB, H), jnp.float32),
        pltpu.VMEM((RPT, H), jnp.float32),
        pltpu.SemaphoreType.DMA,
    ],
    compiler_params=pltpu.CompilerParams(use_tc_tiling_on_sc=False),
  )


def _mm_body(x_ref, w_ref, o_ref):
    o_ref[...] = jnp.dot(x_ref[...], w_ref[...],
                         preferred_element_type=jnp.float32)


_mm_xw1 = pl.pallas_call(
    _mm_body,
    grid=(N // ROW_BLK,),
    in_specs=[pl.BlockSpec((ROW_BLK, D), lambda i: (i, 0)),
              pl.BlockSpec((D, H), lambda i: (0, 0))],
    out_specs=pl.BlockSpec((ROW_BLK, H), lambda i: (i, 0)),
    out_shape=jax.ShapeDtypeStruct((N, H), jnp.float32),
)


def _mid_body(p_ref, b1_ref, w2_ref, o_ref):
    h = p_ref[0] + p_ref[1] + b1_ref[...]
    h = jnp.maximum(h, 0.0)
    o_ref[...] = jnp.dot(h, w2_ref[...], preferred_element_type=jnp.float32)


_mid = pl.pallas_call(
    _mid_body,
    grid=(N // ROW_BLK,),
    in_specs=[pl.BlockSpec((2, ROW_BLK, H), lambda i: (0, i, 0)),
              pl.BlockSpec((1, H), lambda i: (0, 0)),
              pl.BlockSpec((H, H), lambda i: (0, 0))],
    out_specs=pl.BlockSpec((ROW_BLK, H), lambda i: (i, 0)),
    out_shape=jax.ShapeDtypeStruct((N, H), jnp.float32),
)


def _head_body(p_ref, b2_ref, wd_ref, bd_ref, eps_ref, o_ref):
    h2 = p_ref[0] + p_ref[1] + b2_ref[...]
    params = jnp.dot(h2, wd_ref[...],
                     preferred_element_type=jnp.float32) + bd_ref[...]
    eps = eps_ref[...]
    cols = []
    for i in range(LATENT):
        s = params[:, i:i + 1]
        for j in range(i + 1):
            k = i * (i + 1) // 2 + j
            c = params[:, LATENT + k:LATENT + k + 1]
            if i == j:
                c = jax.nn.softplus(c) + 1e-5
            s = s + c * eps[:, j:j + 1]
        cols.append(s)
    o_ref[...] = jnp.concatenate(cols, axis=1)


_head = pl.pallas_call(
    _head_body,
    grid=(N // ROW_BLK,),
    in_specs=[pl.BlockSpec((2, ROW_BLK, H), lambda i: (0, i, 0)),
              pl.BlockSpec((1, H), lambda i: (0, 0)),
              pl.BlockSpec((H, PARAMS), lambda i: (0, 0)),
              pl.BlockSpec((1, PARAMS), lambda i: (0, 0)),
              pl.BlockSpec((ROW_BLK, LATENT), lambda i: (i, 0))],
    out_specs=pl.BlockSpec((ROW_BLK, LATENT), lambda i: (i, 0)),
    out_shape=jax.ShapeDtypeStruct((N, LATENT), jnp.float32),
)


def kernel(x, edge_index, edge_weight, eps, W1, b1, W2, b2, Wd, bd):
    src = edge_index[0].astype(jnp.int32).reshape(NW, NB, SB)
    dst = edge_index[1].astype(jnp.int32).reshape(NW, NB, SB)
    ew = edge_weight.reshape(NW, NB, SB)

    sc_aggregate = _get_sc_aggregate()
    hw1 = _mm_xw1(x, W1)
    p1 = sc_aggregate(hw1, src, dst, ew)[:, :N, :]
    hw2 = _mid(p1, b1.reshape(1, H), W2)
    p2 = sc_aggregate(hw2, src, dst, ew)[:, :N, :]
    return _head(p2, b2.reshape(1, H), Wd, bd.reshape(1, PARAMS), eps)


# speedup vs baseline: 14.7518x; 1.7619x over previous
import functools

import numpy as np
import jax
import jax.numpy as jnp
from jax import lax
from jax.experimental import pallas as pl
from jax.experimental.pallas import tpu as pltpu
from jax.experimental.pallas import tpu_sc as plsc

N = 10000
E = 320000
D = 128
H = 16
LATENT = 7
PARAMS = 35

NC = 2
NS = 16
NW = NC * NS
SB = 80
EW = E // NW
NB = EW // SB
NROWS = E // SB
NPAD = 10240
RPT = NPAD // NS
NBUF = 5
NOUT = NB // NBUF

ROW_BLK = 1000


def _sc_body(hw, srcm, dstm, wm, out, acc, src_v, dst_v, w_v, rows_v, msg_v,
             zb_v, gsem, ssem):
    cid = lax.axis_index("c")
    sid = lax.axis_index("s")
    wid = sid * NC + cid

    def _z(i, c):
        zb_v[i, :] = jnp.zeros((H,), jnp.float32)
        return c

    lax.fori_loop(0, RPT, _z, 0)
    pltpu.sync_copy(zb_v, acc.at[pl.ds(sid * RPT, RPT)])
    plsc.subcore_barrier()

    pltpu.sync_copy(srcm.at[wid], src_v)
    pltpu.sync_copy(dstm.at[wid], dst_v)
    pltpu.sync_copy(wm.at[wid], w_v)

    def _mul(j, b):
        for g in range(SB // H):
            w16 = w_v[j, pl.ds(g * H, H)]
            for e in range(H):
                msg_v[b, g * H + e, :] = rows_v[b, g * H + e, :] * w16[e]

    def _gather(j, b):
        pltpu.async_copy(hw.at[src_v.at[j]], rows_v.at[b], gsem.at[b])

    def _gwait(b):
        pltpu.make_async_copy(hw.at[src_v.at[0]], rows_v.at[b],
                              gsem.at[b]).wait()

    def _scatter(j, b):
        pltpu.async_copy(msg_v.at[b], acc.at[dst_v.at[j]], ssem.at[b],
                         add=True)

    def _swait(j, b):
        pltpu.make_async_copy(msg_v.at[b], acc.at[dst_v.at[j]],
                              ssem.at[b]).wait()

    for b in range(NBUF):
        _gather(b, b)
    for b in range(NBUF):
        _gwait(b)
        _mul(b, b)
        _scatter(b, b)
        _gather(b + NBUF, b)

    def _outer(o, c):
        for b in range(NBUF):
            j = o * NBUF + b
            _gwait(b)
            _swait(j, b)
            _mul(j, b)
            _scatter(j, b)
            _gather(j + NBUF, b)
        return c

    lax.fori_loop(1, NOUT - 1, _outer, 0)

    for b in range(NBUF):
        j = (NOUT - 1) * NBUF + b
        _gwait(b)
        _swait(j, b)
        _mul(j, b)
        _scatter(j, b)
    for b in range(NBUF):
        _swait(0, b)
    plsc.subcore_barrier()

    pltpu.sync_copy(acc.at[pl.ds(sid * RPT, RPT)],
                    out.at[cid, pl.ds(sid * RPT, RPT)])


@functools.lru_cache(maxsize=None)
def _get_sc_aggregate():
  return pl.kernel(
    _sc_body,
    out_type=jax.ShapeDtypeStruct((NC, NPAD, H), jnp.float32),
    mesh=plsc.VectorSubcoreMesh(core_axis_name="c", subcore_axis_name="s",
                                num_cores=NC, num_subcores=NS),
    scratch_types=[
        pltpu.VMEM_SHARED((NPAD, H), jnp.float32),
        pltpu.VMEM((NB, SB), jnp.int32),
        pltpu.VMEM((NB, SB), jnp.int32),
        pltpu.VMEM((NB, SB), jnp.float32),
        pltpu.VMEM((NBUF, SB, H), jnp.float32),
        pltpu.VMEM((NBUF, SB, H), jnp.float32),
        pltpu.VMEM((RPT, H), jnp.float32),
        pltpu.SemaphoreType.DMA((NBUF,)),
        pltpu.SemaphoreType.DMA((NBUF,)),
    ],
    compiler_params=pltpu.CompilerParams(use_tc_tiling_on_sc=False),
  )


def _mm_body(x_ref, w_ref, o_ref):
    o_ref[...] = jnp.dot(x_ref[...], w_ref[...],
                         preferred_element_type=jnp.float32)


_mm_xw1 = pl.pallas_call(
    _mm_body,
    grid=(N // ROW_BLK,),
    in_specs=[pl.BlockSpec((ROW_BLK, D), lambda i: (i, 0)),
              pl.BlockSpec((D, H), lambda i: (0, 0))],
    out_specs=pl.BlockSpec((ROW_BLK, H), lambda i: (i, 0)),
    out_shape=jax.ShapeDtypeStruct((N, H), jnp.float32),
)


def _mid_body(p_ref, b1_ref, w2_ref, o_ref):
    h = p_ref[0] + p_ref[1] + b1_ref[...]
    h = jnp.maximum(h, 0.0)
    o_ref[...] = jnp.dot(h, w2_ref[...], preferred_element_type=jnp.float32)


_mid = pl.pallas_call(
    _mid_body,
    grid=(N // ROW_BLK,),
    in_specs=[pl.BlockSpec((2, ROW_BLK, H), lambda i: (0, i, 0)),
              pl.BlockSpec((1, H), lambda i: (0, 0)),
              pl.BlockSpec((H, H), lambda i: (0, 0))],
    out_specs=pl.BlockSpec((ROW_BLK, H), lambda i: (i, 0)),
    out_shape=jax.ShapeDtypeStruct((N, H), jnp.float32),
)


def _head_body(p_ref, b2_ref, wd_ref, bd_ref, eps_ref, o_ref):
    h2 = p_ref[0] + p_ref[1] + b2_ref[...]
    params = jnp.dot(h2, wd_ref[...],
                     preferred_element_type=jnp.float32) + bd_ref[...]
    eps = eps_ref[...]
    cols = []
    for i in range(LATENT):
        s = params[:, i:i + 1]
        for j in range(i + 1):
            k = i * (i + 1) // 2 + j
            c = params[:, LATENT + k:LATENT + k + 1]
            if i == j:
                c = jax.nn.softplus(c) + 1e-5
            s = s + c * eps[:, j:j + 1]
        cols.append(s)
    o_ref[...] = jnp.concatenate(cols, axis=1)


_head = pl.pallas_call(
    _head_body,
    grid=(N // ROW_BLK,),
    in_specs=[pl.BlockSpec((2, ROW_BLK, H), lambda i: (0, i, 0)),
              pl.BlockSpec((1, H), lambda i: (0, 0)),
              pl.BlockSpec((H, PARAMS), lambda i: (0, 0)),
              pl.BlockSpec((1, PARAMS), lambda i: (0, 0)),
              pl.BlockSpec((ROW_BLK, LATENT), lambda i: (i, 0))],
    out_specs=pl.BlockSpec((ROW_BLK, LATENT), lambda i: (i, 0)),
    out_shape=jax.ShapeDtypeStruct((N, LATENT), jnp.float32),
)


def kernel(x, edge_index, edge_weight, eps, W1, b1, W2, b2, Wd, bd):
    src = edge_index[0].astype(jnp.int32).reshape(NW, NB, SB)
    dst = edge_index[1].astype(jnp.int32).reshape(NW, NB, SB)
    ew = edge_weight.reshape(NW, NB, SB)

    sc_aggregate = _get_sc_aggregate()
    hw1 = _mm_xw1(x, W1)
    p1 = sc_aggregate(hw1, src, dst, ew)[:, :N, :]
    hw2 = _mid(p1, b1.reshape(1, H), W2)
    p2 = sc_aggregate(hw2, src, dst, ew)[:, :N, :]
    return _head(p2, b2.reshape(1, H), Wd, bd.reshape(1, PARAMS), eps)


# speedup vs baseline: 17.6774x; 1.1983x over previous
import functools

import numpy as np
import jax
import jax.numpy as jnp
from jax import lax
from jax.experimental import pallas as pl
from jax.experimental.pallas import tpu as pltpu
from jax.experimental.pallas import tpu_sc as plsc

N = 10000
E = 320000
D = 128
H = 16
LATENT = 7
PARAMS = 35

NC = 2
NS = 16
NW = NC * NS
SB = 80
EW = E // NW
NB = EW // SB
NROWS = E // SB
NPAD = 10240
RPT = NPAD // NS
NBUF = 5
NOUT = NB // NBUF

ROW_BLK = 1000


def _sc_body(hw, srcm, dstm, wm, out, acc, src_v, dst_v, w_v, rows_v, msg_v,
             zb_v, gsem, ssem):
    cid = lax.axis_index("c")
    sid = lax.axis_index("s")
    wid = sid * NC + cid

    def _z(i, c):
        zb_v[i, :] = jnp.zeros((H,), jnp.float32)
        return c

    lax.fori_loop(0, RPT, _z, 0)
    pltpu.sync_copy(zb_v, acc.at[pl.ds(sid * RPT, RPT)])
    plsc.subcore_barrier()

    pltpu.sync_copy(srcm.at[wid], src_v)
    pltpu.sync_copy(dstm.at[wid], dst_v)
    pltpu.sync_copy(wm.at[wid], w_v)

    def _mul(j, b):
        for g in range(SB // H):
            w16 = w_v[j, pl.ds(g * H, H)]
            for e in range(H):
                msg_v[b, g * H + e, :] = rows_v[b, g * H + e, :] * w16[e]

    def _gather(j, b):
        pltpu.async_copy(hw.at[src_v.at[j]], rows_v.at[b], gsem.at[b])

    def _gwait(b):
        pltpu.make_async_copy(hw.at[src_v.at[0]], rows_v.at[b],
                              gsem.at[b]).wait()

    def _scatter(j, b):
        pltpu.async_copy(msg_v.at[b], acc.at[dst_v.at[j]], ssem.at[b],
                         add=True)

    def _swait(j, b):
        pltpu.make_async_copy(msg_v.at[b], acc.at[dst_v.at[j]],
                              ssem.at[b]).wait()

    for b in range(NBUF):
        _gather(b, b)
    for b in range(NBUF):
        _gwait(b)
        _mul(b, b)
        _scatter(b, b)
        _gather(b + NBUF, b)

    def _outer(o, c):
        for b in range(NBUF):
            j = o * NBUF + b
            _gwait(b)
            _swait(j, b)
            _mul(j, b)
            _scatter(j, b)
            _gather(j + NBUF, b)
        return c

    lax.fori_loop(1, NOUT - 1, _outer, 0)

    for b in range(NBUF):
        j = (NOUT - 1) * NBUF + b
        _gwait(b)
        _swait(j, b)
        _mul(j, b)
        _scatter(j, b)
    for b in range(NBUF):
        _swait(0, b)
    plsc.subcore_barrier()

    pltpu.sync_copy(acc.at[pl.ds(sid * RPT, RPT)],
                    out.at[cid, pl.ds(sid * RPT, RPT)])


@functools.lru_cache(maxsize=None)
def _get_sc_aggregate():
  return pl.kernel(
    _sc_body,
    out_type=jax.ShapeDtypeStruct((NC, NPAD, H), jnp.float32),
    mesh=plsc.VectorSubcoreMesh(core_axis_name="c", subcore_axis_name="s",
                                num_cores=NC, num_subcores=NS),
    scratch_types=[
        pltpu.VMEM_SHARED((NPAD, H), jnp.float32),
        pltpu.VMEM((NB, SB), jnp.int32),
        pltpu.VMEM((NB, SB), jnp.int32),
        pltpu.VMEM((NB, SB), jnp.float32),
        pltpu.VMEM((NBUF, SB, H), jnp.float32),
        pltpu.VMEM((NBUF, SB, H), jnp.float32),
        pltpu.VMEM((RPT, H), jnp.float32),
        pltpu.SemaphoreType.DMA((NBUF,)),
        pltpu.SemaphoreType.DMA((NBUF,)),
    ],
    compiler_params=pltpu.CompilerParams(use_tc_tiling_on_sc=False),
  )


def _mm_body(x_ref, w_ref, o_ref):
    o_ref[...] = jnp.dot(x_ref[...], w_ref[...],
                         preferred_element_type=jnp.float32)


_mm_xw1 = pl.pallas_call(
    _mm_body,
    grid=(N // ROW_BLK,),
    in_specs=[pl.BlockSpec((ROW_BLK, D), lambda i: (i, 0)),
              pl.BlockSpec((D, H), lambda i: (0, 0))],
    out_specs=pl.BlockSpec((ROW_BLK, H), lambda i: (i, 0)),
    out_shape=jax.ShapeDtypeStruct((N, H), jnp.float32),
)


def _mid_body(p_ref, b1_ref, w2_ref, o_ref):
    h = p_ref[0] + p_ref[1] + b1_ref[...]
    h = jnp.maximum(h, 0.0)
    o_ref[...] = jnp.dot(h, w2_ref[...], preferred_element_type=jnp.float32)


_mid = pl.pallas_call(
    _mid_body,
    grid=(N // ROW_BLK,),
    in_specs=[pl.BlockSpec((2, ROW_BLK, H), lambda i: (0, i, 0)),
              pl.BlockSpec((1, H), lambda i: (0, 0)),
              pl.BlockSpec((H, H), lambda i: (0, 0))],
    out_specs=pl.BlockSpec((ROW_BLK, H), lambda i: (i, 0)),
    out_shape=jax.ShapeDtypeStruct((N, H), jnp.float32),
)


def _head_body(p_ref, b2_ref, wd_ref, bd_ref, eps_ref, cmat_ref, smat_ref,
               dmask_ref, ones_ref, o_ref):
    h2 = p_ref[0] + p_ref[1] + b2_ref[...]
    params = jnp.dot(h2, wd_ref[...],
                     preferred_element_type=jnp.float32) + bd_ref[...]
    sp = jax.nn.softplus(params) + 1e-5
    dmask = dmask_ref[...]
    pd = dmask * sp + (1.0 - dmask) * params
    g = jnp.dot(eps_ref[...], cmat_ref[...],
                preferred_element_type=jnp.float32,
                precision=jax.lax.Precision.HIGHEST) + ones_ref[...]
    o_ref[...] = jnp.dot(pd * g, smat_ref[...],
                         preferred_element_type=jnp.float32,
                         precision=jax.lax.Precision.HIGHEST)


_head = pl.pallas_call(
    _head_body,
    grid=(N // ROW_BLK,),
    in_specs=[pl.BlockSpec((2, ROW_BLK, H), lambda i: (0, i, 0)),
              pl.BlockSpec((1, H), lambda i: (0, 0)),
              pl.BlockSpec((H, PARAMS), lambda i: (0, 0)),
              pl.BlockSpec((1, PARAMS), lambda i: (0, 0)),
              pl.BlockSpec((ROW_BLK, LATENT), lambda i: (i, 0)),
              pl.BlockSpec((LATENT, PARAMS), lambda i: (0, 0)),
              pl.BlockSpec((PARAMS, LATENT), lambda i: (0, 0)),
              pl.BlockSpec((1, PARAMS), lambda i: (0, 0)),
              pl.BlockSpec((1, PARAMS), lambda i: (0, 0))],
    out_specs=pl.BlockSpec((ROW_BLK, LATENT), lambda i: (i, 0)),
    out_shape=jax.ShapeDtypeStruct((N, LATENT), jnp.float32),
)

_RI, _CI = np.tril_indices(LATENT)
_CMAT = np.zeros((LATENT, PARAMS), np.float32)
_SMAT = np.zeros((PARAMS, LATENT), np.float32)
_DMASK = np.zeros((1, PARAMS), np.float32)
_ONES = np.zeros((1, PARAMS), np.float32)
_ONES[0, :LATENT] = 1.0
for _k in range(len(_RI)):
    _CMAT[_CI[_k], LATENT + _k] = 1.0
    _SMAT[LATENT + _k, _RI[_k]] = 1.0
    if _RI[_k] == _CI[_k]:
        _DMASK[0, LATENT + _k] = 1.0
for _i in range(LATENT):
    _SMAT[_i, _i] = 1.0


def kernel(x, edge_index, edge_weight, eps, W1, b1, W2, b2, Wd, bd):
    src = edge_index[0].astype(jnp.int32).reshape(NW, NB, SB)
    dst = edge_index[1].astype(jnp.int32).reshape(NW, NB, SB)
    ew = edge_weight.reshape(NW, NB, SB)

    sc_aggregate = _get_sc_aggregate()
    hw1 = _mm_xw1(x, W1)
    p1 = sc_aggregate(hw1, src, dst, ew)
    hw2 = _mid(p1, b1.reshape(1, H), W2)
    p2 = sc_aggregate(hw2, src, dst, ew)
    return _head(p2, b2.reshape(1, H), Wd, bd.reshape(1, PARAMS), eps,
                 jnp.asarray(_CMAT), jnp.asarray(_SMAT),
                 jnp.asarray(_DMASK), jnp.asarray(_ONES))
